# baseline (speedup 1.0000x reference)
import functools

import jax
import jax.numpy as jnp
from jax import lax
from jax.experimental import pallas as pl
from jax.experimental.pallas import tpu as pltpu
from jax.experimental.pallas import tpu_sc as plsc

M = 2097152
D = 16
NC, NS, L = 2, 16, 16
NW = NC * NS
NG = 2
WPG = NW // NG
WC = M // WPG
CC = 2048
NCHUNK = WC // CC
NBUF = 2

_mesh = plsc.VectorSubcoreMesh(core_axis_name="c", subcore_axis_name="s")


@functools.partial(
    pl.kernel,
    mesh=_mesh,
    out_type=jax.ShapeDtypeStruct((D, M), jnp.float32),
    compiler_params=pltpu.CompilerParams(use_tc_tiling_on_sc=True),
    scratch_types=[
        pltpu.VMEM((NBUF, 8, CC), jnp.float32),
        pltpu.VMEM((NBUF, 8, CC), jnp.float32),
        pltpu.VMEM((NBUF, 8, CC), jnp.float32),
        pltpu.SemaphoreType.DMA((NBUF,)),
        pltpu.SemaphoreType.DMA((NBUF,)),
        pltpu.SemaphoreType.DMA((NBUF,)),
    ],
)
def _sc_masked_add(a_hbm, b_hbm, o_hbm, a_v, b_v, o_v, la_sem, lb_sem, st_sem):
    wid = lax.axis_index("s") * NC + lax.axis_index("c")
    g = wid // WPG
    base = pl.multiple_of((wid % WPG) * WC, CC)

    def rows(hbm, off):
        return hbm.at[pl.ds(g * 8, 8), pl.ds(off, CC)]

    def load(ci, p):
        off = pl.multiple_of(base + ci * CC, CC)
        pltpu.async_copy(rows(a_hbm, off), a_v.at[p], la_sem.at[p])
        pltpu.async_copy(rows(b_hbm, off), b_v.at[p], lb_sem.at[p])

    for p in range(NBUF):
        load(p, p)

    def pair_body(it, carry):
        for p in range(NBUF):
            ci = it * NBUF + p
            pltpu.make_async_copy(rows(a_hbm, base), a_v.at[p], la_sem.at[p]).wait()
            pltpu.make_async_copy(rows(b_hbm, base), b_v.at[p], lb_sem.at[p]).wait()

            @pl.when(ci >= NBUF)
            def _():
                pltpu.make_async_copy(
                    o_v.at[p], rows(o_hbm, base), st_sem.at[p]).wait()

            @plsc.parallel_loop(0, CC // L, unroll=4)
            def _(j):
                s = pl.ds(j * L, L)
                for r in range(8):
                    a = a_v[p, r, s]
                    b = b_v[p, r, s]
                    o_v[p, r, s] = jnp.where(a == 1.0, 0.0, a) + b

            off = pl.multiple_of(base + ci * CC, CC)
            pltpu.async_copy(o_v.at[p], rows(o_hbm, off), st_sem.at[p])

            @pl.when(ci + NBUF < NCHUNK)
            def _():
                load(ci + NBUF, p)

        return carry

    lax.fori_loop(0, NCHUNK // NBUF, pair_body, 0)

    for p in range(NBUF):
        pltpu.make_async_copy(o_v.at[p], rows(o_hbm, base), st_sem.at[p]).wait()


def kernel(x_1, x_2):
    out = _sc_masked_add(x_1.T, x_2.T)
    return out.T


# speedup vs baseline: 1.2838x; 1.2838x over previous
import jax
import jax.numpy as jnp
from jax.experimental import pallas as pl

BN = 131072


def _body(a_ref, b_ref, o_ref):
    a = a_ref[...]
    o_ref[...] = jnp.where(a == 1.0, 0.0, a) + b_ref[...]


def kernel(x_1, x_2):
    a = x_1.T
    b = x_2.T
    n = a.shape[1]
    out = pl.pallas_call(
        _body,
        grid=(n // BN,),
        in_specs=[
            pl.BlockSpec((16, BN), lambda i: (0, i)),
            pl.BlockSpec((16, BN), lambda i: (0, i)),
        ],
        out_specs=pl.BlockSpec((16, BN), lambda i: (0, i)),
        out_shape=jax.ShapeDtypeStruct((16, n), jnp.float32),
    )(a, b)
    return out.T
